# baseline (speedup 1.0000x reference)
import jax
import jax.numpy as jnp
from jax.experimental import pallas as pl

NUM_GRAPHS = 64
DIMS_K = [(128, 16, 48, 16), (16, 32, 24, 32), (32, 64, 12, 64), (64, 128, 6, 128), (128, 256, 3, None)]


def _bn(x, g, b):
    m = jnp.mean(x, axis=0)
    v = jnp.var(x, axis=0)
    return (x - m) / jnp.sqrt(v + 1e-5) * g + b


def _gat_conv(x, src, dst, edge_attr, p, H, C):
    N = x.shape[0]
    deg = jax.ops.segment_sum(jnp.ones_like(dst, dtype=jnp.float32), dst, num_segments=N)
    loop_attr = jax.ops.segment_sum(edge_attr, dst, num_segments=N) / jnp.maximum(deg, 1.0)[:, None]
    sl = jnp.arange(N, dtype=src.dtype)
    src2 = jnp.concatenate([src, sl])
    dst2 = jnp.concatenate([dst, sl])
    ea = jnp.concatenate([edge_attr, loop_attr], axis=0)
    xw = (x @ p['W']).reshape(N, H, C)
    a_s = jnp.sum(xw * p['att_src'], axis=-1)
    a_d = jnp.sum(xw * p['att_dst'], axis=-1)
    ew = (ea @ p['W_edge']).reshape(-1, H, C)
    a_e = jnp.sum(ew * p['att_edge'], axis=-1)
    alpha = jax.nn.leaky_relu(a_s[src2] + a_d[dst2] + a_e, 0.2)
    amax = jax.ops.segment_max(alpha, dst2, num_segments=N)
    alpha = jnp.exp(alpha - amax[dst2])
    denom = jax.ops.segment_sum(alpha, dst2, num_segments=N)
    alpha = alpha / (denom[dst2] + 1e-16)
    out = jax.ops.segment_sum(xw[src2] * alpha[..., None], dst2, num_segments=N)
    return out.reshape(N, H * C) + p['bias']


def _head_kernel(pooled_ref, w_ref, b_ref, o_ref):
    o_ref[...] = pooled_ref[...] @ w_ref[...] + b_ref[...]


def kernel(x, edge_index, edge_attr, batch, params):
    src = edge_index[0]
    dst = edge_index[1]
    for (fin, C, H, lin), p in zip(DIMS_K, params['blocks']):
        x = _gat_conv(x, src, dst, edge_attr, p, H, C)
        x = jax.nn.relu(_bn(x, p['bn0_g'], p['bn0_b']))
        if lin is not None:
            x = x @ p['lin_w'] + p['lin_b']
            x = jax.nn.relu(_bn(x, p['bn1_g'], p['bn1_b']))
    counts = jax.ops.segment_sum(jnp.ones((x.shape[0],), jnp.float32), batch, num_segments=NUM_GRAPHS)
    pooled = jax.ops.segment_sum(x, batch, num_segments=NUM_GRAPHS) / jnp.maximum(counts, 1.0)[:, None]
    out = pl.pallas_call(
        _head_kernel,
        out_shape=jax.ShapeDtypeStruct((NUM_GRAPHS, params['head_w'].shape[1]), jnp.float32),
    )(pooled, params['head_w'], params['head_b'][None, :])
    return out


# speedup vs baseline: 11.4144x; 11.4144x over previous
import functools

import jax
import jax.numpy as jnp
from jax import lax
from jax.experimental import pallas as pl
from jax.experimental.pallas import tpu as pltpu
from jax.experimental.pallas import tpu_sc as plsc

N_NODES = 10000
N_EDGES = 160000
NUM_GRAPHS = 64
D = 768
LAYER_DIMS = [(128, 16, 48, 16), (16, 32, 24, 32), (32, 64, 12, 64), (64, 128, 6, 128), (128, 256, 3, None)]

NW = 32
NT = 320
NP = NW * NT
CN = 32
NCH = NT // CN
EB = 64
DW = 896
EPAD = ((N_EDGES + EB) // 1024 + 1) * 1024

F32 = jnp.float32
I32 = jnp.int32


def _hp(h):
    return ((h + 15) // 16) * 16


def _nodemm_body(apply_act, fin, x_ref, wcat_ref, wd_ref, s_ref, t_ref, xw_ref, ad_ref):
    i = pl.program_id(0)
    x = x_ref[...]
    if apply_act:
        x = jnp.maximum(x * s_ref[...] + t_ref[...], 0.0)
    rid = i * x.shape[0] + lax.broadcasted_iota(I32, (x.shape[0], 1), 0)
    x = jnp.where(rid < N_NODES, x, 0.0)
    xw_ref[...] = jnp.dot(x, wcat_ref[...], preferred_element_type=F32)
    ad_ref[...] = jnp.dot(x, wd_ref[...], preferred_element_type=F32)


def _nodemm(x, wcat, wd, s, t, apply_act):
    fin = x.shape[1]
    dw = wcat.shape[1]
    hp = wd.shape[1]
    bn = 256
    return pl.pallas_call(
        functools.partial(_nodemm_body, apply_act, fin),
        grid=(NP // bn,),
        in_specs=[
            pl.BlockSpec((bn, fin), lambda i: (i, 0)),
            pl.BlockSpec((fin, dw), lambda i: (0, 0)),
            pl.BlockSpec((fin, hp), lambda i: (0, 0)),
            pl.BlockSpec((1, fin), lambda i: (0, 0)),
            pl.BlockSpec((1, fin), lambda i: (0, 0)),
        ],
        out_specs=[
            pl.BlockSpec((bn, dw), lambda i: (i, 0)),
            pl.BlockSpec((bn, hp), lambda i: (i, 0)),
        ],
        out_shape=[
            jax.ShapeDtypeStruct((NP, dw), F32),
            jax.ShapeDtypeStruct((NP, hp), F32),
        ],
    )(x, wcat, wd, s, t)


def _edgemm_body(ea_ref, wf_ref, ae_ref):
    ae_ref[...] = jnp.dot(ea_ref[...], wf_ref[...], preferred_element_type=F32)


def _edgemm(ea, wf):
    hp = wf.shape[1]
    bn = 1024
    return pl.pallas_call(
        _edgemm_body,
        grid=(EPAD // bn,),
        in_specs=[
            pl.BlockSpec((bn, 3), lambda i: (i, 0)),
            pl.BlockSpec((3, hp), lambda i: (0, 0)),
        ],
        out_specs=pl.BlockSpec((bn, hp), lambda i: (i, 0)),
        out_shape=jax.ShapeDtypeStruct((EPAD, hp), F32),
    )(ea, wf)


def _stats_body(nblk, arr_ref, g_ref, b_ref, s_ref, t_ref, acc1, acc2):
    i = pl.program_id(0)

    @pl.when(i == 0)
    def _():
        acc1[...] = jnp.zeros_like(acc1)
        acc2[...] = jnp.zeros_like(acc2)

    a = arr_ref[...]
    acc1[...] += jnp.sum(a, axis=0, keepdims=True)
    acc2[...] += jnp.sum(a * a, axis=0, keepdims=True)

    @pl.when(i == nblk - 1)
    def _():
        n = jnp.float32(N_NODES)
        mean = acc1[...] / n
        var = acc2[...] / n - mean * mean
        s = g_ref[...] * lax.rsqrt(var + 1e-5)
        s_ref[...] = s
        t_ref[...] = b_ref[...] - mean * s


def _stats(arr, g, b):
    dx = arr.shape[1]
    bn = 512
    nblk = NP // bn
    return pl.pallas_call(
        functools.partial(_stats_body, nblk),
        grid=(nblk,),
        in_specs=[
            pl.BlockSpec((bn, dx), lambda i: (i, 0)),
            pl.BlockSpec((1, dx), lambda i: (0, 0)),
            pl.BlockSpec((1, dx), lambda i: (0, 0)),
        ],
        out_specs=[
            pl.BlockSpec((1, dx), lambda i: (0, 0)),
            pl.BlockSpec((1, dx), lambda i: (0, 0)),
        ],
        out_shape=[
            jax.ShapeDtypeStruct((1, dx), F32),
            jax.ShapeDtypeStruct((1, dx), F32),
        ],
        scratch_shapes=[
            pltpu.VMEM((1, dx), F32),
            pltpu.VMEM((1, dx), F32),
        ],
    )(arr, g, b)


def _linmm_body(agg_ref, s_ref, t_ref, w_ref, b_ref, z_ref):
    i = pl.program_id(0)
    h = jnp.maximum(agg_ref[...] * s_ref[...] + t_ref[...], 0.0)
    rid = i * h.shape[0] + lax.broadcasted_iota(I32, (h.shape[0], 1), 0)
    h = jnp.where(rid < N_NODES, h, 0.0)
    z = jnp.dot(h, w_ref[...], preferred_element_type=F32) + b_ref[...]
    z_ref[...] = jnp.where(rid < N_NODES, z, 0.0)


def _linmm(agg, s, t, w, b):
    lin = w.shape[1]
    bn = 512
    return pl.pallas_call(
        _linmm_body,
        grid=(NP // bn,),
        in_specs=[
            pl.BlockSpec((bn, D), lambda i: (i, 0)),
            pl.BlockSpec((1, D), lambda i: (0, 0)),
            pl.BlockSpec((1, D), lambda i: (0, 0)),
            pl.BlockSpec((D, lin), lambda i: (0, 0)),
            pl.BlockSpec((1, lin), lambda i: (0, 0)),
        ],
        out_specs=pl.BlockSpec((bn, lin), lambda i: (i, 0)),
        out_shape=jax.ShapeDtypeStruct((NP, lin), F32),
    )(agg, s, t, w, b)


def _pool_body(agg_ref, s_ref, t_ref, batch_ref, p_ref, c_ref):
    i = pl.program_id(0)

    @pl.when(i == 0)
    def _():
        p_ref[...] = jnp.zeros_like(p_ref)
        c_ref[...] = jnp.zeros_like(c_ref)

    x = jnp.maximum(agg_ref[...] * s_ref[...] + t_ref[...], 0.0)
    bn = x.shape[0]
    rid = i * bn + lax.broadcasted_iota(I32, (bn, 1), 0)
    valid = rid < N_NODES
    x = jnp.where(valid, x, 0.0)
    bvec = batch_ref[0, 0, :]
    gids = lax.broadcasted_iota(I32, (NUM_GRAPHS, bn), 0)
    oh = jnp.where((bvec[None, :] == gids) & (valid[:, 0])[None, :], 1.0, 0.0)
    p_ref[...] += jnp.dot(oh, x, preferred_element_type=F32)
    c_ref[...] += jnp.dot(oh, jnp.ones((bn, 128), F32), preferred_element_type=F32)


def _pool(agg, s, t, batch3d):
    bn = 512
    return pl.pallas_call(
        _pool_body,
        grid=(NP // bn,),
        in_specs=[
            pl.BlockSpec((bn, D), lambda i: (i, 0)),
            pl.BlockSpec((1, D), lambda i: (0, 0)),
            pl.BlockSpec((1, D), lambda i: (0, 0)),
            pl.BlockSpec((1, 1, bn), lambda i: (i, 0, 0)),
        ],
        out_specs=[
            pl.BlockSpec((NUM_GRAPHS, D), lambda i: (0, 0)),
            pl.BlockSpec((NUM_GRAPHS, 128), lambda i: (0, 0)),
        ],
        out_shape=[
            jax.ShapeDtypeStruct((NUM_GRAPHS, D), F32),
            jax.ShapeDtypeStruct((NUM_GRAPHS, 128), F32),
        ],
    )(agg, s, t, batch3d)


def _head_body(p_ref, c_ref, w_ref, b_ref, o_ref):
    cnt = jnp.maximum(c_ref[...][:, 0:1], 1.0)
    o_ref[...] = jnp.dot(p_ref[...] / cnt, w_ref[...], preferred_element_type=F32) + b_ref[...]


def _head(pooled, cnt, w, b):
    return pl.pallas_call(
        _head_body,
        out_shape=jax.ShapeDtypeStruct((NUM_GRAPHS, w.shape[1]), F32),
    )(pooled, cnt, w, b)


def _sc_agg_body(hp, dw, cc, xw_hbm, ad_hbm, ae_hbm, src_hbm, dst_hbm, rp_hbm,
                 out_hbm, rows_v, self_v, acc_v, den_v, aes_v, adv, ae_v,
                 idx_v, dst_v, rp_v, sem):
    nh = hp // 16
    wid = lax.axis_index("c") * 16 + lax.axis_index("s")
    base_n = pl.multiple_of(wid * NT, 8)
    pltpu.sync_copy(rp_hbm.at[pl.ds(base_n, NT + 32)], rp_v)

    def chunk_body(c, _):
        nbl = c * CN
        nb = base_n + nbl

        def zero_body(i, _):
            z16 = jnp.zeros((16,), F32)
            for j in range(D // 16):
                acc_v[i, pl.ds(j * 16, 16)] = z16
            for k in range(nh):
                den_v[i, pl.ds(k * 16, 16)] = z16
                aes_v[i, pl.ds(k * 16, 16)] = z16
            return 0

        lax.fori_loop(0, CN, zero_body, 0)
        pltpu.sync_copy(ad_hbm.at[pl.ds(nb, CN)], adv)
        e0 = rp_v[pl.ds(nbl, 16)][0]
        e1 = rp_v[pl.ds(nbl + CN, 16)][0]
        base_e = e0 - lax.rem(e0, 8)
        nbat = lax.div(e1 - base_e + (EB - 1), EB)

        def batch_body(b, _):
            ebase = pl.multiple_of(base_e + b * EB, 8)
            pltpu.sync_copy(src_hbm.at[pl.ds(ebase, EB)], idx_v)
            pltpu.sync_copy(dst_hbm.at[pl.ds(ebase, EB)], dst_v.at[pl.ds(0, EB)])
            pltpu.sync_copy(ae_hbm.at[pl.ds(ebase, EB)], ae_v)
            pltpu.async_copy(xw_hbm.at[idx_v], rows_v, sem).wait()
            lo = jnp.maximum(e0, ebase) - ebase
            hi = jnp.minimum(e1, ebase + EB) - ebase

            def edge_body(i, _):
                dl = dst_v[pl.ds(i, 16)][0] - nb
                ws = []
                for k in range(nh):
                    a = rows_v[i, pl.ds(D + k * 16, 16)]
                    ad_k = adv[dl, pl.ds(k * 16, 16)]
                    ae_k = ae_v[i, pl.ds(k * 16, 16)]
                    lg = a + ad_k + ae_k
                    lg = jnp.where(lg < 0, lg * 0.2, lg)
                    w = jnp.exp(lg)
                    ws.append(w)
                    plsc.addupdate(den_v.at[dl, pl.ds(k * 16, 16)], w)
                    plsc.addupdate(aes_v.at[dl, pl.ds(k * 16, 16)], ae_k)
                for j in range(D // 16):
                    h = (j * 16) // cc
                    wsc = ws[h // 16][h % 16]
                    plsc.addupdate(acc_v.at[dl, pl.ds(j * 16, 16)],
                                   wsc * rows_v[i, pl.ds(j * 16, 16)])
                return 0

            lax.fori_loop(lo, hi, edge_body, 0)
            return 0

        lax.fori_loop(0, nbat, batch_body, 0)

        for g in range(CN // 16):
            pltpu.sync_copy(xw_hbm.at[pl.ds(nb + g * 16, 16)], self_v)

            def node_body(u, _):
                i = g * 16 + u
                rp2 = rp_v[pl.ds(nbl + i, 16)]
                deg = rp2[1] - rp2[0]
                invd = 1.0 / jnp.maximum(jnp.full((16,), deg.astype(F32)), 1.0)
                ws = []
                rs = []
                for k in range(nh):
                    a = self_v[u, pl.ds(D + k * 16, 16)]
                    ad_k = adv[i, pl.ds(k * 16, 16)]
                    ael = aes_v[i, pl.ds(k * 16, 16)] * invd
                    lg = a + ad_k + ael
                    lg = jnp.where(lg < 0, lg * 0.2, lg)
                    w = jnp.exp(lg)
                    ws.append(w)
                    rs.append(1.0 / (den_v[i, pl.ds(k * 16, 16)] + w))
                for j in range(D // 16):
                    h = (j * 16) // cc
                    wsc = ws[h // 16][h % 16]
                    rsc = rs[h // 16][h % 16]
                    acc_v[i, pl.ds(j * 16, 16)] = (
                        acc_v[i, pl.ds(j * 16, 16)]
                        + wsc * self_v[u, pl.ds(j * 16, 16)]) * rsc
                return 0

            lax.fori_loop(0, 16, node_body, 0)

        pltpu.sync_copy(acc_v, out_hbm.at[pl.ds(nb, CN)])
        return 0

    lax.fori_loop(0, NCH, chunk_body, 0)


def _sc_agg(xw_cat, ad, ae, srcs, dsts, row_ptr, hp, cc):
    dw = DW
    mesh = plsc.VectorSubcoreMesh(core_axis_name="c", subcore_axis_name="s",
                                  num_cores=2, num_subcores=16)
    fn = pl.kernel(
        functools.partial(_sc_agg_body, hp, dw, cc),
        out_type=jax.ShapeDtypeStruct((NP, D), F32),
        mesh=mesh,
        scratch_types=[
            pltpu.VMEM((EB, dw), F32),
            pltpu.VMEM((16, dw), F32),
            pltpu.VMEM((CN, D), F32),
            pltpu.VMEM((CN, hp), F32),
            pltpu.VMEM((CN, hp), F32),
            pltpu.VMEM((CN, hp), F32),
            pltpu.VMEM((EB, hp), F32),
            pltpu.VMEM((EB,), I32),
            pltpu.VMEM((EB + 16,), I32),
            pltpu.VMEM((NT + 32,), I32),
            pltpu.SemaphoreType.DMA,
        ],
    )
    return fn(xw_cat, ad, ae, srcs, dsts, row_ptr)


def _fold_weights(p, fin, c, h):
    hp = _hp(h)
    w = p['W']
    ws = jnp.einsum('fhc,hc->fh', w.reshape(fin, h, c), p['att_src'][0])
    wd = jnp.einsum('fhc,hc->fh', w.reshape(fin, h, c), p['att_dst'][0])
    wf = jnp.einsum('khc,hc->kh', p['W_edge'].reshape(3, h, c), p['att_edge'][0])
    pad = hp - h
    ws = jnp.pad(ws, ((0, 0), (0, pad)))
    wd = jnp.pad(wd, ((0, 0), (0, pad)))
    wf = jnp.pad(wf, ((0, 0), (0, pad)))
    wcat = jnp.concatenate(
        [w, ws, jnp.zeros((fin, DW - D - hp), F32)], axis=1)
    return wcat, wd, wf


def kernel(x, edge_index, edge_attr, batch, params):
    src = edge_index[0]
    dst = edge_index[1]

    perm = jnp.argsort(dst)
    srcs = jnp.concatenate([src[perm], jnp.zeros((EB,), I32)])
    dsts = jnp.concatenate([dst[perm], jnp.zeros((EB,), I32)])
    ea_s = jnp.concatenate(
        [edge_attr[perm], jnp.zeros((EPAD - N_EDGES, 3), F32)], axis=0)
    row_ptr = jnp.searchsorted(
        dst[perm], jnp.arange(NP + 64, dtype=I32), side='left').astype(I32)

    xin = jnp.pad(x, ((0, NP - N_NODES), (0, 0)))
    batch3d = jnp.pad(batch, (0, NP - N_NODES)).reshape(NP // 512, 1, 512)

    s_aff = jnp.ones((1, x.shape[1]), F32)
    t_aff = jnp.zeros((1, x.shape[1]), F32)
    agg = None
    for li, ((fin, c, h, lin), p) in enumerate(zip(LAYER_DIMS, params['blocks'])):
        hp = _hp(h)
        wcat, wd, wf = _fold_weights(p, fin, c, h)
        xw_cat, ad = _nodemm(xin, wcat, wd, s_aff, t_aff, apply_act=(li > 0))
        ae = _edgemm(ea_s, wf)
        agg = _sc_agg(xw_cat, ad, ae, srcs, dsts, row_ptr, hp, c)
        s0, t0 = _stats(agg, p['bn0_g'][None, :], p['bn0_b'][None, :])
        if lin is not None:
            z = _linmm(agg, s0, t0, p['lin_w'], p['lin_b'][None, :])
            s_aff, t_aff = _stats(z, p['bn1_g'][None, :], p['bn1_b'][None, :])
            xin = z
        else:
            pooled, cnt = _pool(agg, s0, t0, batch3d)
            return _head(pooled, cnt, params['head_w'], params['head_b'][None, :])


# speedup vs baseline: 21.3394x; 1.8695x over previous
import functools

import jax
import jax.numpy as jnp
from jax import lax
from jax.experimental import pallas as pl
from jax.experimental.pallas import tpu as pltpu
from jax.experimental.pallas import tpu_sc as plsc

N_NODES = 10000
N_EDGES = 160000
NUM_GRAPHS = 64
D = 768
LAYER_DIMS = [(128, 16, 48, 16), (16, 32, 24, 32), (32, 64, 12, 64), (64, 128, 6, 128), (128, 256, 3, None)]

NW = 32
NT = 320
NP = NW * NT
CN = 32
NCH = NT // CN
EB = 40
DW = 896
EPAD = ((N_EDGES + EB) // 1024 + 1) * 1024

F32 = jnp.float32
I32 = jnp.int32


def _hp(h):
    return ((h + 15) // 16) * 16


def _nodemm_body(apply_act, fin, x_ref, wcat_ref, wd_ref, s_ref, t_ref, xw_ref, ad_ref):
    i = pl.program_id(0)
    x = x_ref[...]
    if apply_act:
        x = jnp.maximum(x * s_ref[...] + t_ref[...], 0.0)
    rid = i * x.shape[0] + lax.broadcasted_iota(I32, (x.shape[0], 1), 0)
    x = jnp.where(rid < N_NODES, x, 0.0)
    xw_ref[...] = jnp.dot(x, wcat_ref[...], preferred_element_type=F32)
    ad_ref[...] = jnp.dot(x, wd_ref[...], preferred_element_type=F32)


def _nodemm(x, wcat, wd, s, t, apply_act):
    fin = x.shape[1]
    dw = wcat.shape[1]
    hp = wd.shape[1]
    bn = 256
    return pl.pallas_call(
        functools.partial(_nodemm_body, apply_act, fin),
        grid=(NP // bn,),
        in_specs=[
            pl.BlockSpec((bn, fin), lambda i: (i, 0)),
            pl.BlockSpec((fin, dw), lambda i: (0, 0)),
            pl.BlockSpec((fin, hp), lambda i: (0, 0)),
            pl.BlockSpec((1, fin), lambda i: (0, 0)),
            pl.BlockSpec((1, fin), lambda i: (0, 0)),
        ],
        out_specs=[
            pl.BlockSpec((bn, dw), lambda i: (i, 0)),
            pl.BlockSpec((bn, hp), lambda i: (i, 0)),
        ],
        out_shape=[
            jax.ShapeDtypeStruct((NP, dw), F32),
            jax.ShapeDtypeStruct((NP, hp), F32),
        ],
    )(x, wcat, wd, s, t)


def _edgemm_body(ea_ref, wf_ref, ae_ref):
    ae_ref[...] = jnp.dot(ea_ref[...], wf_ref[...], preferred_element_type=F32)


def _edgemm(ea, wf):
    hp = wf.shape[1]
    bn = 1024
    return pl.pallas_call(
        _edgemm_body,
        grid=(EPAD // bn,),
        in_specs=[
            pl.BlockSpec((bn, 3), lambda i: (i, 0)),
            pl.BlockSpec((3, hp), lambda i: (0, 0)),
        ],
        out_specs=pl.BlockSpec((bn, hp), lambda i: (i, 0)),
        out_shape=jax.ShapeDtypeStruct((EPAD, hp), F32),
    )(ea, wf)


def _stats_body(nblk, arr_ref, g_ref, b_ref, s_ref, t_ref, acc1, acc2):
    i = pl.program_id(0)

    @pl.when(i == 0)
    def _():
        acc1[...] = jnp.zeros_like(acc1)
        acc2[...] = jnp.zeros_like(acc2)

    a = arr_ref[...]
    acc1[...] += jnp.sum(a, axis=0, keepdims=True)
    acc2[...] += jnp.sum(a * a, axis=0, keepdims=True)

    @pl.when(i == nblk - 1)
    def _():
        n = jnp.float32(N_NODES)
        mean = acc1[...] / n
        var = acc2[...] / n - mean * mean
        s = g_ref[...] * lax.rsqrt(var + 1e-5)
        s_ref[...] = s
        t_ref[...] = b_ref[...] - mean * s


def _stats(arr, g, b):
    dx = arr.shape[1]
    bn = 512
    nblk = NP // bn
    return pl.pallas_call(
        functools.partial(_stats_body, nblk),
        grid=(nblk,),
        in_specs=[
            pl.BlockSpec((bn, dx), lambda i: (i, 0)),
            pl.BlockSpec((1, dx), lambda i: (0, 0)),
            pl.BlockSpec((1, dx), lambda i: (0, 0)),
        ],
        out_specs=[
            pl.BlockSpec((1, dx), lambda i: (0, 0)),
            pl.BlockSpec((1, dx), lambda i: (0, 0)),
        ],
        out_shape=[
            jax.ShapeDtypeStruct((1, dx), F32),
            jax.ShapeDtypeStruct((1, dx), F32),
        ],
        scratch_shapes=[
            pltpu.VMEM((1, dx), F32),
            pltpu.VMEM((1, dx), F32),
        ],
    )(arr, g, b)


def _linmm_body(agg_ref, s_ref, t_ref, w_ref, b_ref, z_ref):
    i = pl.program_id(0)
    h = jnp.maximum(agg_ref[...] * s_ref[...] + t_ref[...], 0.0)
    rid = i * h.shape[0] + lax.broadcasted_iota(I32, (h.shape[0], 1), 0)
    h = jnp.where(rid < N_NODES, h, 0.0)
    z = jnp.dot(h, w_ref[...], preferred_element_type=F32) + b_ref[...]
    z_ref[...] = jnp.where(rid < N_NODES, z, 0.0)


def _linmm(agg, s, t, w, b):
    lin = w.shape[1]
    bn = 512
    return pl.pallas_call(
        _linmm_body,
        grid=(NP // bn,),
        in_specs=[
            pl.BlockSpec((bn, D), lambda i: (i, 0)),
            pl.BlockSpec((1, D), lambda i: (0, 0)),
            pl.BlockSpec((1, D), lambda i: (0, 0)),
            pl.BlockSpec((D, lin), lambda i: (0, 0)),
            pl.BlockSpec((1, lin), lambda i: (0, 0)),
        ],
        out_specs=pl.BlockSpec((bn, lin), lambda i: (i, 0)),
        out_shape=jax.ShapeDtypeStruct((NP, lin), F32),
    )(agg, s, t, w, b)


def _pool_body(agg_ref, s_ref, t_ref, batch_ref, p_ref, c_ref):
    i = pl.program_id(0)

    @pl.when(i == 0)
    def _():
        p_ref[...] = jnp.zeros_like(p_ref)
        c_ref[...] = jnp.zeros_like(c_ref)

    x = jnp.maximum(agg_ref[...] * s_ref[...] + t_ref[...], 0.0)
    bn = x.shape[0]
    rid = i * bn + lax.broadcasted_iota(I32, (bn, 1), 0)
    valid = rid < N_NODES
    x = jnp.where(valid, x, 0.0)
    bvec = batch_ref[0, 0, :]
    gids = lax.broadcasted_iota(I32, (NUM_GRAPHS, bn), 0)
    oh = jnp.where((bvec[None, :] == gids) & (valid[:, 0])[None, :], 1.0, 0.0)
    p_ref[...] += jnp.dot(oh, x, preferred_element_type=F32)
    c_ref[...] += jnp.dot(oh, jnp.ones((bn, 128), F32), preferred_element_type=F32)


def _pool(agg, s, t, batch3d):
    bn = 512
    return pl.pallas_call(
        _pool_body,
        grid=(NP // bn,),
        in_specs=[
            pl.BlockSpec((bn, D), lambda i: (i, 0)),
            pl.BlockSpec((1, D), lambda i: (0, 0)),
            pl.BlockSpec((1, D), lambda i: (0, 0)),
            pl.BlockSpec((1, 1, bn), lambda i: (i, 0, 0)),
        ],
        out_specs=[
            pl.BlockSpec((NUM_GRAPHS, D), lambda i: (0, 0)),
            pl.BlockSpec((NUM_GRAPHS, 128), lambda i: (0, 0)),
        ],
        out_shape=[
            jax.ShapeDtypeStruct((NUM_GRAPHS, D), F32),
            jax.ShapeDtypeStruct((NUM_GRAPHS, 128), F32),
        ],
    )(agg, s, t, batch3d)


def _head_body(p_ref, c_ref, w_ref, b_ref, o_ref):
    cnt = jnp.maximum(c_ref[...][:, 0:1], 1.0)
    o_ref[...] = jnp.dot(p_ref[...] / cnt, w_ref[...], preferred_element_type=F32) + b_ref[...]


def _head(pooled, cnt, w, b):
    return pl.pallas_call(
        _head_body,
        out_shape=jax.ShapeDtypeStruct((NUM_GRAPHS, w.shape[1]), F32),
    )(pooled, cnt, w, b)


def _sc_agg_body(hp, dw, cc, xw_hbm, ad_hbm, ae_hbm, src_hbm, dst_hbm, rp_hbm,
                 out_hbm, rows_v, acc_v, adv, ae_v, idx_v, dst_v, rp_v,
                 sem0, sem1):
    nh = hp // 16
    njf = D // 16
    njt = njf + 2 * nh
    sems = (sem0, sem1)
    wid = lax.axis_index("c") * 16 + lax.axis_index("s")
    base_n = pl.multiple_of(wid * NT, 8)
    pltpu.sync_copy(rp_hbm.at[pl.ds(base_n, NT + 32)], rp_v)

    def chunk_body(c, _):
        nbl = c * CN
        nb = base_n + nbl

        def zero_body(i, _):
            z16 = jnp.zeros((16,), F32)
            for j in range(njt):
                acc_v[i, pl.ds(j * 16, 16)] = z16
            return 0

        lax.fori_loop(0, CN, zero_body, 0)
        pltpu.sync_copy(ad_hbm.at[pl.ds(nb, CN)], adv)
        e0 = rp_v[pl.ds(nbl, 16)][0]
        e1 = rp_v[pl.ds(nbl + CN, 16)][0]
        base_e = e0 - lax.rem(e0, 8)
        nbat = lax.div(e1 - base_e + (EB - 1), EB)

        def issue(b, half):
            ebase = pl.multiple_of(base_e + b * EB, 8)
            ho = half * EB
            pltpu.sync_copy(src_hbm.at[pl.ds(ebase, EB)],
                            idx_v.at[pl.ds(ho, EB)])
            pltpu.sync_copy(dst_hbm.at[pl.ds(ebase, EB)],
                            dst_v.at[pl.ds(ho, EB)])
            pltpu.sync_copy(ae_hbm.at[pl.ds(ebase, EB)],
                            ae_v.at[pl.ds(ho, EB)])
            pltpu.async_copy(xw_hbm.at[idx_v.at[pl.ds(ho, EB)]],
                             rows_v.at[pl.ds(ho, EB)], sems[half])

        def wait(half):
            ho = half * EB
            pltpu.make_async_copy(xw_hbm.at[idx_v.at[pl.ds(ho, EB)]],
                                  rows_v.at[pl.ds(ho, EB)], sems[half]).wait()

        def process(b, half):
            ebase = pl.multiple_of(base_e + b * EB, 8)
            lo = jnp.maximum(e0, ebase) - ebase
            hi = jnp.minimum(e1, ebase + EB) - ebase
            b0 = half * EB

            def loop_a(i, _):
                r = b0 + i
                dl = dst_v[pl.ds(r, 16)][0] - nb
                ws = []
                for k in range(nh):
                    a = rows_v[r, pl.ds(D + k * 16, 16)]
                    ad_k = adv[dl, pl.ds(k * 16, 16)]
                    ae_k = ae_v[r, pl.ds(k * 16, 16)]
                    lg = a + ad_k + ae_k
                    lg = jnp.where(lg < 0, lg * 0.2, lg)
                    w = jnp.exp(lg)
                    ws.append(w)
                    rows_v[r, pl.ds(D + k * 16, 16)] = w
                    rows_v[r, pl.ds(D + hp + k * 16, 16)] = ae_k
                for j in range(njf):
                    h = (j * 16) // cc
                    wsc = ws[h // 16][h % 16]
                    rows_v[r, pl.ds(j * 16, 16)] = (
                        wsc * rows_v[r, pl.ds(j * 16, 16)])
                return 0

            lax.fori_loop(lo, hi, loop_a, 0)

            nfirst = dst_v[pl.ds(b0 + lo, 16)][0] - nb
            nlast = dst_v[pl.ds(b0 + hi - 1, 16)][0] - nb

            def loop_b(n, _):
                rpv = rp_v[pl.ds(nbl + n, 16)]
                il = jnp.maximum(rpv[0], ebase + lo) - ebase
                ih = jnp.minimum(rpv[1], ebase + hi) - ebase
                for go in range(0, njt, 9):
                    gl = min(9, njt - go)
                    vs = tuple(acc_v[n, pl.ds((go + q) * 16, 16)]
                               for q in range(gl))

                    def sum_body(i, vv):
                        return tuple(
                            v + rows_v[b0 + i, pl.ds((go + q) * 16, 16)]
                            for q, v in enumerate(vv))

                    vs = lax.fori_loop(il, ih, sum_body, vs)
                    for q in range(gl):
                        acc_v[n, pl.ds((go + q) * 16, 16)] = vs[q]
                return 0

            lax.fori_loop(nfirst, nlast + 1, loop_b, 0)

        @pl.when(nbat > 0)
        def _():
            issue(0, 0)

        def pair_body(p, _):
            b0 = 2 * p

            @pl.when(b0 + 1 < nbat)
            def _():
                issue(b0 + 1, 1)

            wait(0)
            process(b0, 0)

            @pl.when(b0 + 1 < nbat)
            def _():
                @pl.when(b0 + 2 < nbat)
                def _():
                    issue(b0 + 2, 0)

                wait(1)
                process(b0 + 1, 1)

            return 0

        lax.fori_loop(0, lax.div(nbat + 1, 2), pair_body, 0)

        for g in range(CN // 16):
            pltpu.sync_copy(xw_hbm.at[pl.ds(nb + g * 16, 16)],
                            rows_v.at[pl.ds(0, 16)])

            def node_body(u, _):
                i = g * 16 + u
                rp2 = rp_v[pl.ds(nbl + i, 16)]
                deg = rp2[1] - rp2[0]
                invd = 1.0 / jnp.maximum(jnp.full((16,), deg.astype(F32)), 1.0)
                ws = []
                rs = []
                for k in range(nh):
                    a = rows_v[u, pl.ds(D + k * 16, 16)]
                    ad_k = adv[i, pl.ds(k * 16, 16)]
                    ael = acc_v[i, pl.ds(D + hp + k * 16, 16)] * invd
                    lg = a + ad_k + ael
                    lg = jnp.where(lg < 0, lg * 0.2, lg)
                    w = jnp.exp(lg)
                    ws.append(w)
                    rs.append(1.0 / (acc_v[i, pl.ds(D + k * 16, 16)] + w))
                for j in range(njf):
                    h = (j * 16) // cc
                    wsc = ws[h // 16][h % 16]
                    rsc = rs[h // 16][h % 16]
                    acc_v[i, pl.ds(j * 16, 16)] = (
                        acc_v[i, pl.ds(j * 16, 16)]
                        + wsc * rows_v[u, pl.ds(j * 16, 16)]) * rsc
                return 0

            lax.fori_loop(0, 16, node_body, 0)

        pltpu.sync_copy(acc_v.at[:, pl.ds(0, D)], out_hbm.at[pl.ds(nb, CN)])
        return 0

    lax.fori_loop(0, NCH, chunk_body, 0)


def _sc_agg(xw_cat, ad, ae, srcs, dsts, row_ptr, hp, cc):
    dw = DW
    mesh = plsc.VectorSubcoreMesh(core_axis_name="c", subcore_axis_name="s",
                                  num_cores=2, num_subcores=16)
    fn = pl.kernel(
        functools.partial(_sc_agg_body, hp, dw, cc),
        out_type=jax.ShapeDtypeStruct((NP, D), F32),
        mesh=mesh,
        scratch_types=[
            pltpu.VMEM((2 * EB, dw), F32),
            pltpu.VMEM((CN, D + 2 * hp), F32),
            pltpu.VMEM((CN, hp), F32),
            pltpu.VMEM((2 * EB, hp), F32),
            pltpu.VMEM((2 * EB,), I32),
            pltpu.VMEM((2 * EB + 16,), I32),
            pltpu.VMEM((NT + 32,), I32),
            pltpu.SemaphoreType.DMA,
            pltpu.SemaphoreType.DMA,
        ],
    )
    return fn(xw_cat, ad, ae, srcs, dsts, row_ptr)


def _fold_weights(p, fin, c, h):
    hp = _hp(h)
    w = p['W']
    ws = jnp.einsum('fhc,hc->fh', w.reshape(fin, h, c), p['att_src'][0])
    wd = jnp.einsum('fhc,hc->fh', w.reshape(fin, h, c), p['att_dst'][0])
    wf = jnp.einsum('khc,hc->kh', p['W_edge'].reshape(3, h, c), p['att_edge'][0])
    pad = hp - h
    ws = jnp.pad(ws, ((0, 0), (0, pad)))
    wd = jnp.pad(wd, ((0, 0), (0, pad)))
    wf = jnp.pad(wf, ((0, 0), (0, pad)))
    wcat = jnp.concatenate(
        [w, ws, jnp.zeros((fin, DW - D - hp), F32)], axis=1)
    return wcat, wd, wf


def kernel(x, edge_index, edge_attr, batch, params):
    src = edge_index[0]
    dst = edge_index[1]

    perm = jnp.argsort(dst)
    srcs = jnp.concatenate([src[perm], jnp.zeros((EB,), I32)])
    dsts = jnp.concatenate([dst[perm], jnp.zeros((EB,), I32)])
    ea_s = jnp.concatenate(
        [edge_attr[perm], jnp.zeros((EPAD - N_EDGES, 3), F32)], axis=0)
    row_ptr = jnp.searchsorted(
        dst[perm], jnp.arange(NP + 64, dtype=I32), side='left').astype(I32)

    xin = jnp.pad(x, ((0, NP - N_NODES), (0, 0)))
    batch3d = jnp.pad(batch, (0, NP - N_NODES)).reshape(NP // 512, 1, 512)

    s_aff = jnp.ones((1, x.shape[1]), F32)
    t_aff = jnp.zeros((1, x.shape[1]), F32)
    agg = None
    for li, ((fin, c, h, lin), p) in enumerate(zip(LAYER_DIMS, params['blocks'])):
        hp = _hp(h)
        wcat, wd, wf = _fold_weights(p, fin, c, h)
        xw_cat, ad = _nodemm(xin, wcat, wd, s_aff, t_aff, apply_act=(li > 0))
        ae = _edgemm(ea_s, wf)
        agg = _sc_agg(xw_cat, ad, ae, srcs, dsts, row_ptr, hp, c)
        s0, t0 = _stats(agg, p['bn0_g'][None, :], p['bn0_b'][None, :])
        if lin is not None:
            z = _linmm(agg, s0, t0, p['lin_w'], p['lin_b'][None, :])
            s_aff, t_aff = _stats(z, p['bn1_g'][None, :], p['bn1_b'][None, :])
            xin = z
        else:
            pooled, cnt = _pool(agg, s0, t0, batch3d)
            return _head(pooled, cnt, params['head_w'], params['head_b'][None, :])


# speedup vs baseline: 23.3784x; 1.0956x over previous
import functools

import jax
import jax.numpy as jnp
from jax import lax
from jax.experimental import pallas as pl
from jax.experimental.pallas import tpu as pltpu
from jax.experimental.pallas import tpu_sc as plsc

N_NODES = 10000
N_EDGES = 160000
NUM_GRAPHS = 64
D = 768
LAYER_DIMS = [(128, 16, 48, 16), (16, 32, 24, 32), (32, 64, 12, 64), (64, 128, 6, 128), (128, 256, 3, None)]

NW = 32
NT = 320
NP = NW * NT
CN = 32
NCH = NT // CN
EB = 40
DW = 896
EPAD = ((N_EDGES + EB) // 1024 + 1) * 1024

F32 = jnp.float32
I32 = jnp.int32


def _hp(h):
    return ((h + 15) // 16) * 16


def _nodemm_body(apply_act, fin, x_ref, wcat_ref, wd_ref, s_ref, t_ref, xw_ref, ad_ref):
    i = pl.program_id(0)
    x = x_ref[...]
    if apply_act:
        x = jnp.maximum(x * s_ref[...] + t_ref[...], 0.0)
    rid = i * x.shape[0] + lax.broadcasted_iota(I32, (x.shape[0], 1), 0)
    x = jnp.where(rid < N_NODES, x, 0.0)
    xw_ref[...] = jnp.dot(x, wcat_ref[...], preferred_element_type=F32)
    ad_ref[...] = jnp.dot(x, wd_ref[...], preferred_element_type=F32)


def _nodemm(x, wcat, wd, s, t, apply_act):
    fin = x.shape[1]
    dw = wcat.shape[1]
    hp = wd.shape[1]
    bn = 256
    return pl.pallas_call(
        functools.partial(_nodemm_body, apply_act, fin),
        grid=(NP // bn,),
        in_specs=[
            pl.BlockSpec((bn, fin), lambda i: (i, 0)),
            pl.BlockSpec((fin, dw), lambda i: (0, 0)),
            pl.BlockSpec((fin, hp), lambda i: (0, 0)),
            pl.BlockSpec((1, fin), lambda i: (0, 0)),
            pl.BlockSpec((1, fin), lambda i: (0, 0)),
        ],
        out_specs=[
            pl.BlockSpec((bn, dw), lambda i: (i, 0)),
            pl.BlockSpec((bn, hp), lambda i: (i, 0)),
        ],
        out_shape=[
            jax.ShapeDtypeStruct((NP, dw), F32),
            jax.ShapeDtypeStruct((NP, hp), F32),
        ],
    )(x, wcat, wd, s, t)


def _edgemm_body(ea_ref, wf_ref, ae_ref):
    ae_ref[...] = jnp.dot(ea_ref[...], wf_ref[...], preferred_element_type=F32)


def _edgemm(ea, wf):
    hp = wf.shape[1]
    bn = 1024
    return pl.pallas_call(
        _edgemm_body,
        grid=(EPAD // bn,),
        in_specs=[
            pl.BlockSpec((bn, 3), lambda i: (i, 0)),
            pl.BlockSpec((3, hp), lambda i: (0, 0)),
        ],
        out_specs=pl.BlockSpec((bn, hp), lambda i: (i, 0)),
        out_shape=jax.ShapeDtypeStruct((EPAD, hp), F32),
    )(ea, wf)


def _stats_body(nblk, arr_ref, g_ref, b_ref, s_ref, t_ref, acc1, acc2):
    i = pl.program_id(0)

    @pl.when(i == 0)
    def _():
        acc1[...] = jnp.zeros_like(acc1)
        acc2[...] = jnp.zeros_like(acc2)

    a = arr_ref[...]
    acc1[...] += jnp.sum(a, axis=0, keepdims=True)
    acc2[...] += jnp.sum(a * a, axis=0, keepdims=True)

    @pl.when(i == nblk - 1)
    def _():
        n = jnp.float32(N_NODES)
        mean = acc1[...] / n
        var = acc2[...] / n - mean * mean
        s = g_ref[...] * lax.rsqrt(var + 1e-5)
        s_ref[...] = s
        t_ref[...] = b_ref[...] - mean * s


def _stats(arr, g, b):
    dx = arr.shape[1]
    bn = 512
    nblk = NP // bn
    return pl.pallas_call(
        functools.partial(_stats_body, nblk),
        grid=(nblk,),
        in_specs=[
            pl.BlockSpec((bn, dx), lambda i: (i, 0)),
            pl.BlockSpec((1, dx), lambda i: (0, 0)),
            pl.BlockSpec((1, dx), lambda i: (0, 0)),
        ],
        out_specs=[
            pl.BlockSpec((1, dx), lambda i: (0, 0)),
            pl.BlockSpec((1, dx), lambda i: (0, 0)),
        ],
        out_shape=[
            jax.ShapeDtypeStruct((1, dx), F32),
            jax.ShapeDtypeStruct((1, dx), F32),
        ],
        scratch_shapes=[
            pltpu.VMEM((1, dx), F32),
            pltpu.VMEM((1, dx), F32),
        ],
    )(arr, g, b)


def _linmm_body(agg_ref, s_ref, t_ref, w_ref, b_ref, z_ref):
    i = pl.program_id(0)
    h = jnp.maximum(agg_ref[...] * s_ref[...] + t_ref[...], 0.0)
    rid = i * h.shape[0] + lax.broadcasted_iota(I32, (h.shape[0], 1), 0)
    h = jnp.where(rid < N_NODES, h, 0.0)
    z = jnp.dot(h, w_ref[...], preferred_element_type=F32) + b_ref[...]
    z_ref[...] = jnp.where(rid < N_NODES, z, 0.0)


def _linmm(agg, s, t, w, b):
    lin = w.shape[1]
    bn = 512
    return pl.pallas_call(
        _linmm_body,
        grid=(NP // bn,),
        in_specs=[
            pl.BlockSpec((bn, D), lambda i: (i, 0)),
            pl.BlockSpec((1, D), lambda i: (0, 0)),
            pl.BlockSpec((1, D), lambda i: (0, 0)),
            pl.BlockSpec((D, lin), lambda i: (0, 0)),
            pl.BlockSpec((1, lin), lambda i: (0, 0)),
        ],
        out_specs=pl.BlockSpec((bn, lin), lambda i: (i, 0)),
        out_shape=jax.ShapeDtypeStruct((NP, lin), F32),
    )(agg, s, t, w, b)


def _pool_body(agg_ref, s_ref, t_ref, batch_ref, p_ref, c_ref):
    i = pl.program_id(0)

    @pl.when(i == 0)
    def _():
        p_ref[...] = jnp.zeros_like(p_ref)
        c_ref[...] = jnp.zeros_like(c_ref)

    x = jnp.maximum(agg_ref[...] * s_ref[...] + t_ref[...], 0.0)
    bn = x.shape[0]
    rid = i * bn + lax.broadcasted_iota(I32, (bn, 1), 0)
    valid = rid < N_NODES
    x = jnp.where(valid, x, 0.0)
    bvec = batch_ref[0, 0, :]
    gids = lax.broadcasted_iota(I32, (NUM_GRAPHS, bn), 0)
    oh = jnp.where((bvec[None, :] == gids) & (valid[:, 0])[None, :], 1.0, 0.0)
    p_ref[...] += jnp.dot(oh, x, preferred_element_type=F32)
    c_ref[...] += jnp.dot(oh, jnp.ones((bn, 128), F32), preferred_element_type=F32)


def _pool(agg, s, t, batch3d):
    bn = 512
    return pl.pallas_call(
        _pool_body,
        grid=(NP // bn,),
        in_specs=[
            pl.BlockSpec((bn, D), lambda i: (i, 0)),
            pl.BlockSpec((1, D), lambda i: (0, 0)),
            pl.BlockSpec((1, D), lambda i: (0, 0)),
            pl.BlockSpec((1, 1, bn), lambda i: (i, 0, 0)),
        ],
        out_specs=[
            pl.BlockSpec((NUM_GRAPHS, D), lambda i: (0, 0)),
            pl.BlockSpec((NUM_GRAPHS, 128), lambda i: (0, 0)),
        ],
        out_shape=[
            jax.ShapeDtypeStruct((NUM_GRAPHS, D), F32),
            jax.ShapeDtypeStruct((NUM_GRAPHS, 128), F32),
        ],
    )(agg, s, t, batch3d)


def _head_body(p_ref, c_ref, w_ref, b_ref, o_ref):
    cnt = jnp.maximum(c_ref[...][:, 0:1], 1.0)
    o_ref[...] = jnp.dot(p_ref[...] / cnt, w_ref[...], preferred_element_type=F32) + b_ref[...]


def _head(pooled, cnt, w, b):
    return pl.pallas_call(
        _head_body,
        out_shape=jax.ShapeDtypeStruct((NUM_GRAPHS, w.shape[1]), F32),
    )(pooled, cnt, w, b)


def _sc_agg_body(hp, dw, cc, xw_hbm, ad_hbm, ae_hbm, src_hbm, dst_hbm, rp_hbm,
                 out_hbm, rows_v, acc_v, adv, ae_v, idx_v, dst_v, rp_v,
                 sem0, sem1):
    nh = hp // 16
    njf = D // 16
    njt = njf + 2 * nh
    sems = (sem0, sem1)
    wid = lax.axis_index("c") * 16 + lax.axis_index("s")
    base_n = pl.multiple_of(wid * NT, 8)
    pltpu.sync_copy(rp_hbm.at[pl.ds(base_n, NT + 32)], rp_v)

    def chunk_body(c, _):
        nbl = c * CN
        nb = base_n + nbl

        def zero_body(i, _):
            z16 = jnp.zeros((16,), F32)
            for j in range(njt):
                acc_v[i, pl.ds(j * 16, 16)] = z16
            return 0

        lax.fori_loop(0, CN, zero_body, 0)
        pltpu.sync_copy(ad_hbm.at[pl.ds(nb, CN)], adv)
        e0 = rp_v[pl.ds(nbl, 16)][0]
        e1 = rp_v[pl.ds(nbl + CN, 16)][0]
        base_e = e0 - lax.rem(e0, 8)
        nbat = lax.div(e1 - base_e + (EB - 1), EB)

        def issue(b, half):
            ebase = pl.multiple_of(base_e + b * EB, 8)
            ho = half * EB
            pltpu.sync_copy(src_hbm.at[pl.ds(ebase, EB)],
                            idx_v.at[pl.ds(ho, EB)])
            pltpu.sync_copy(dst_hbm.at[pl.ds(ebase, EB)],
                            dst_v.at[pl.ds(ho, EB)])
            pltpu.sync_copy(ae_hbm.at[pl.ds(ebase, EB)],
                            ae_v.at[pl.ds(ho, EB)])
            pltpu.async_copy(xw_hbm.at[idx_v.at[pl.ds(ho, EB)]],
                             rows_v.at[pl.ds(ho, EB)], sems[half])

        def wait(half):
            ho = half * EB
            pltpu.make_async_copy(xw_hbm.at[idx_v.at[pl.ds(ho, EB)]],
                                  rows_v.at[pl.ds(ho, EB)], sems[half]).wait()

        def process(b, half):
            ebase = pl.multiple_of(base_e + b * EB, 8)
            lo = jnp.maximum(e0, ebase) - ebase
            hi = jnp.minimum(e1, ebase + EB) - ebase
            b0 = half * EB

            def loop_a(i, _):
                r = b0 + i
                dl = dst_v[pl.ds(r, 16)][0] - nb
                for k in range(nh):
                    a = rows_v[r, pl.ds(D + k * 16, 16)]
                    ad_k = adv[dl, pl.ds(k * 16, 16)]
                    ae_k = ae_v[r, pl.ds(k * 16, 16)]
                    lg = a + ad_k + ae_k
                    lg = jnp.where(lg < 0, lg * 0.2, lg)
                    w = jnp.exp(lg)
                    rows_v[r, pl.ds(D + k * 16, 16)] = w
                    rows_v[r, pl.ds(D + hp + k * 16, 16)] = ae_k
                return 0

            lax.fori_loop(lo, hi, loop_a, 0)

            nfirst = dst_v[pl.ds(b0 + lo, 16)][0] - nb
            nlast = dst_v[pl.ds(b0 + hi - 1, 16)][0] - nb

            def loop_b(n, _):
                rpv = rp_v[pl.ds(nbl + n, 16)]
                il = jnp.maximum(rpv[0], ebase + lo) - ebase
                ih = jnp.minimum(rpv[1], ebase + hi) - ebase
                for go in range(0, njt, 9):
                    gl = min(9, njt - go)
                    vs = tuple(acc_v[n, pl.ds((go + q) * 16, 16)]
                               for q in range(gl))

                    def sum_body(i, vv):
                        r = b0 + i
                        wv = {}
                        out = []
                        for q, v in enumerate(vv):
                            j = go + q
                            row = rows_v[r, pl.ds(j * 16, 16)]
                            if j < njf:
                                h = (j * 16) // cc
                                k = h // 16
                                if k not in wv:
                                    wv[k] = rows_v[r, pl.ds(D + k * 16, 16)]
                                out.append(v + wv[k][h % 16] * row)
                            else:
                                out.append(v + row)
                        return tuple(out)

                    vs = lax.fori_loop(il, ih, sum_body, vs)
                    for q in range(gl):
                        acc_v[n, pl.ds((go + q) * 16, 16)] = vs[q]
                return 0

            lax.fori_loop(nfirst, nlast + 1, loop_b, 0)

        @pl.when(nbat > 0)
        def _():
            issue(0, 0)

        def pair_body(p, _):
            b0 = 2 * p

            @pl.when(b0 + 1 < nbat)
            def _():
                issue(b0 + 1, 1)

            wait(0)
            process(b0, 0)

            @pl.when(b0 + 1 < nbat)
            def _():
                @pl.when(b0 + 2 < nbat)
                def _():
                    issue(b0 + 2, 0)

                wait(1)
                process(b0 + 1, 1)

            return 0

        lax.fori_loop(0, lax.div(nbat + 1, 2), pair_body, 0)

        for g in range(CN // 16):
            pltpu.sync_copy(xw_hbm.at[pl.ds(nb + g * 16, 16)],
                            rows_v.at[pl.ds(0, 16)])

            def node_body(u, _):
                i = g * 16 + u
                rp2 = rp_v[pl.ds(nbl + i, 16)]
                deg = rp2[1] - rp2[0]
                invd = 1.0 / jnp.maximum(jnp.full((16,), deg.astype(F32)), 1.0)
                ws = []
                rs = []
                for k in range(nh):
                    a = rows_v[u, pl.ds(D + k * 16, 16)]
                    ad_k = adv[i, pl.ds(k * 16, 16)]
                    ael = acc_v[i, pl.ds(D + hp + k * 16, 16)] * invd
                    lg = a + ad_k + ael
                    lg = jnp.where(lg < 0, lg * 0.2, lg)
                    w = jnp.exp(lg)
                    ws.append(w)
                    rs.append(1.0 / (acc_v[i, pl.ds(D + k * 16, 16)] + w))
                for j in range(njf):
                    h = (j * 16) // cc
                    wsc = ws[h // 16][h % 16]
                    rsc = rs[h // 16][h % 16]
                    acc_v[i, pl.ds(j * 16, 16)] = (
                        acc_v[i, pl.ds(j * 16, 16)]
                        + wsc * rows_v[u, pl.ds(j * 16, 16)]) * rsc
                return 0

            lax.fori_loop(0, 16, node_body, 0)

        pltpu.sync_copy(acc_v.at[:, pl.ds(0, D)], out_hbm.at[pl.ds(nb, CN)])
        return 0

    lax.fori_loop(0, NCH, chunk_body, 0)


def _sc_agg(xw_cat, ad, ae, srcs, dsts, row_ptr, hp, cc):
    dw = DW
    mesh = plsc.VectorSubcoreMesh(core_axis_name="c", subcore_axis_name="s",
                                  num_cores=2, num_subcores=16)
    fn = pl.kernel(
        functools.partial(_sc_agg_body, hp, dw, cc),
        out_type=jax.ShapeDtypeStruct((NP, D), F32),
        mesh=mesh,
        scratch_types=[
            pltpu.VMEM((2 * EB, dw), F32),
            pltpu.VMEM((CN, D + 2 * hp), F32),
            pltpu.VMEM((CN, hp), F32),
            pltpu.VMEM((2 * EB, hp), F32),
            pltpu.VMEM((2 * EB,), I32),
            pltpu.VMEM((2 * EB + 16,), I32),
            pltpu.VMEM((NT + 32,), I32),
            pltpu.SemaphoreType.DMA,
            pltpu.SemaphoreType.DMA,
        ],
    )
    return fn(xw_cat, ad, ae, srcs, dsts, row_ptr)


def _fold_weights(p, fin, c, h):
    hp = _hp(h)
    w = p['W']
    ws = jnp.einsum('fhc,hc->fh', w.reshape(fin, h, c), p['att_src'][0])
    wd = jnp.einsum('fhc,hc->fh', w.reshape(fin, h, c), p['att_dst'][0])
    wf = jnp.einsum('khc,hc->kh', p['W_edge'].reshape(3, h, c), p['att_edge'][0])
    pad = hp - h
    ws = jnp.pad(ws, ((0, 0), (0, pad)))
    wd = jnp.pad(wd, ((0, 0), (0, pad)))
    wf = jnp.pad(wf, ((0, 0), (0, pad)))
    wcat = jnp.concatenate(
        [w, ws, jnp.zeros((fin, DW - D - hp), F32)], axis=1)
    return wcat, wd, wf


def kernel(x, edge_index, edge_attr, batch, params):
    src = edge_index[0]
    dst = edge_index[1]

    perm = jnp.argsort(dst)
    srcs = jnp.concatenate([src[perm], jnp.zeros((EB,), I32)])
    dsts = jnp.concatenate([dst[perm], jnp.zeros((EB,), I32)])
    ea_s = jnp.concatenate(
        [edge_attr[perm], jnp.zeros((EPAD - N_EDGES, 3), F32)], axis=0)
    row_ptr = jnp.searchsorted(
        dst[perm], jnp.arange(NP + 64, dtype=I32), side='left').astype(I32)

    xin = jnp.pad(x, ((0, NP - N_NODES), (0, 0)))
    batch3d = jnp.pad(batch, (0, NP - N_NODES)).reshape(NP // 512, 1, 512)

    s_aff = jnp.ones((1, x.shape[1]), F32)
    t_aff = jnp.zeros((1, x.shape[1]), F32)
    agg = None
    for li, ((fin, c, h, lin), p) in enumerate(zip(LAYER_DIMS, params['blocks'])):
        hp = _hp(h)
        wcat, wd, wf = _fold_weights(p, fin, c, h)
        xw_cat, ad = _nodemm(xin, wcat, wd, s_aff, t_aff, apply_act=(li > 0))
        ae = _edgemm(ea_s, wf)
        agg = _sc_agg(xw_cat, ad, ae, srcs, dsts, row_ptr, hp, c)
        s0, t0 = _stats(agg, p['bn0_g'][None, :], p['bn0_b'][None, :])
        if lin is not None:
            z = _linmm(agg, s0, t0, p['lin_w'], p['lin_b'][None, :])
            s_aff, t_aff = _stats(z, p['bn1_g'][None, :], p['bn1_b'][None, :])
            xin = z
        else:
            pooled, cnt = _pool(agg, s0, t0, batch3d)
            return _head(pooled, cnt, params['head_w'], params['head_b'][None, :])


# speedup vs baseline: 23.7750x; 1.0170x over previous
import functools

import jax
import jax.numpy as jnp
from jax import lax
from jax.experimental import pallas as pl
from jax.experimental.pallas import tpu as pltpu
from jax.experimental.pallas import tpu_sc as plsc

N_NODES = 10000
N_EDGES = 160000
NUM_GRAPHS = 64
D = 768
LAYER_DIMS = [(128, 16, 48, 16), (16, 32, 24, 32), (32, 64, 12, 64), (64, 128, 6, 128), (128, 256, 3, None)]

NW = 32
NT = 320
NP = NW * NT
CN = 32
NCH = NT // CN
EB = 40
DW = 896
EPAD = ((N_EDGES + EB) // 1024 + 1) * 1024

F32 = jnp.float32
I32 = jnp.int32


def _hp(h):
    return ((h + 15) // 16) * 16


def _nodemm_body(apply_act, fin, x_ref, wcat_ref, wd_ref, s_ref, t_ref, xw_ref, ad_ref):
    i = pl.program_id(0)
    x = x_ref[...]
    if apply_act:
        x = jnp.maximum(x * s_ref[...] + t_ref[...], 0.0)
    rid = i * x.shape[0] + lax.broadcasted_iota(I32, (x.shape[0], 1), 0)
    x = jnp.where(rid < N_NODES, x, 0.0)
    xw_ref[...] = jnp.dot(x, wcat_ref[...], preferred_element_type=F32)
    ad_ref[...] = jnp.dot(x, wd_ref[...], preferred_element_type=F32)


def _nodemm(x, wcat, wd, s, t, apply_act):
    fin = x.shape[1]
    dw = wcat.shape[1]
    hp = wd.shape[1]
    bn = 256
    return pl.pallas_call(
        functools.partial(_nodemm_body, apply_act, fin),
        grid=(NP // bn,),
        in_specs=[
            pl.BlockSpec((bn, fin), lambda i: (i, 0)),
            pl.BlockSpec((fin, dw), lambda i: (0, 0)),
            pl.BlockSpec((fin, hp), lambda i: (0, 0)),
            pl.BlockSpec((1, fin), lambda i: (0, 0)),
            pl.BlockSpec((1, fin), lambda i: (0, 0)),
        ],
        out_specs=[
            pl.BlockSpec((bn, dw), lambda i: (i, 0)),
            pl.BlockSpec((bn, hp), lambda i: (i, 0)),
        ],
        out_shape=[
            jax.ShapeDtypeStruct((NP, dw), F32),
            jax.ShapeDtypeStruct((NP, hp), F32),
        ],
    )(x, wcat, wd, s, t)


def _edgemm_body(ea_ref, wf_ref, ae_ref):
    ae_ref[...] = jnp.dot(ea_ref[...], wf_ref[...], preferred_element_type=F32)


def _edgemm(ea, wf):
    hp = wf.shape[1]
    bn = 1024
    return pl.pallas_call(
        _edgemm_body,
        grid=(EPAD // bn,),
        in_specs=[
            pl.BlockSpec((bn, 3), lambda i: (i, 0)),
            pl.BlockSpec((3, hp), lambda i: (0, 0)),
        ],
        out_specs=pl.BlockSpec((bn, hp), lambda i: (i, 0)),
        out_shape=jax.ShapeDtypeStruct((EPAD, hp), F32),
    )(ea, wf)


def _stats_body(nblk, arr_ref, g_ref, b_ref, s_ref, t_ref, acc1, acc2):
    i = pl.program_id(0)

    @pl.when(i == 0)
    def _():
        acc1[...] = jnp.zeros_like(acc1)
        acc2[...] = jnp.zeros_like(acc2)

    a = arr_ref[...]
    acc1[...] += jnp.sum(a, axis=0, keepdims=True)
    acc2[...] += jnp.sum(a * a, axis=0, keepdims=True)

    @pl.when(i == nblk - 1)
    def _():
        n = jnp.float32(N_NODES)
        mean = acc1[...] / n
        var = acc2[...] / n - mean * mean
        s = g_ref[...] * lax.rsqrt(var + 1e-5)
        s_ref[...] = s
        t_ref[...] = b_ref[...] - mean * s


def _stats(arr, g, b):
    dx = arr.shape[1]
    bn = 512
    nblk = NP // bn
    return pl.pallas_call(
        functools.partial(_stats_body, nblk),
        grid=(nblk,),
        in_specs=[
            pl.BlockSpec((bn, dx), lambda i: (i, 0)),
            pl.BlockSpec((1, dx), lambda i: (0, 0)),
            pl.BlockSpec((1, dx), lambda i: (0, 0)),
        ],
        out_specs=[
            pl.BlockSpec((1, dx), lambda i: (0, 0)),
            pl.BlockSpec((1, dx), lambda i: (0, 0)),
        ],
        out_shape=[
            jax.ShapeDtypeStruct((1, dx), F32),
            jax.ShapeDtypeStruct((1, dx), F32),
        ],
        scratch_shapes=[
            pltpu.VMEM((1, dx), F32),
            pltpu.VMEM((1, dx), F32),
        ],
    )(arr, g, b)


def _linmm_body(agg_ref, s_ref, t_ref, w_ref, b_ref, z_ref):
    i = pl.program_id(0)
    h = jnp.maximum(agg_ref[...] * s_ref[...] + t_ref[...], 0.0)
    rid = i * h.shape[0] + lax.broadcasted_iota(I32, (h.shape[0], 1), 0)
    h = jnp.where(rid < N_NODES, h, 0.0)
    z = jnp.dot(h, w_ref[...], preferred_element_type=F32) + b_ref[...]
    z_ref[...] = jnp.where(rid < N_NODES, z, 0.0)


def _linmm(agg, s, t, w, b):
    lin = w.shape[1]
    bn = 512
    return pl.pallas_call(
        _linmm_body,
        grid=(NP // bn,),
        in_specs=[
            pl.BlockSpec((bn, D), lambda i: (i, 0)),
            pl.BlockSpec((1, D), lambda i: (0, 0)),
            pl.BlockSpec((1, D), lambda i: (0, 0)),
            pl.BlockSpec((D, lin), lambda i: (0, 0)),
            pl.BlockSpec((1, lin), lambda i: (0, 0)),
        ],
        out_specs=pl.BlockSpec((bn, lin), lambda i: (i, 0)),
        out_shape=jax.ShapeDtypeStruct((NP, lin), F32),
    )(agg, s, t, w, b)


def _pool_body(agg_ref, s_ref, t_ref, batch_ref, p_ref, c_ref):
    i = pl.program_id(0)

    @pl.when(i == 0)
    def _():
        p_ref[...] = jnp.zeros_like(p_ref)
        c_ref[...] = jnp.zeros_like(c_ref)

    x = jnp.maximum(agg_ref[...] * s_ref[...] + t_ref[...], 0.0)
    bn = x.shape[0]
    rid = i * bn + lax.broadcasted_iota(I32, (bn, 1), 0)
    valid = rid < N_NODES
    x = jnp.where(valid, x, 0.0)
    bvec = batch_ref[0, 0, :]
    gids = lax.broadcasted_iota(I32, (NUM_GRAPHS, bn), 0)
    oh = jnp.where((bvec[None, :] == gids) & (valid[:, 0])[None, :], 1.0, 0.0)
    p_ref[...] += jnp.dot(oh, x, preferred_element_type=F32)
    c_ref[...] += jnp.dot(oh, jnp.ones((bn, 128), F32), preferred_element_type=F32)


def _pool(agg, s, t, batch3d):
    bn = 512
    return pl.pallas_call(
        _pool_body,
        grid=(NP // bn,),
        in_specs=[
            pl.BlockSpec((bn, D), lambda i: (i, 0)),
            pl.BlockSpec((1, D), lambda i: (0, 0)),
            pl.BlockSpec((1, D), lambda i: (0, 0)),
            pl.BlockSpec((1, 1, bn), lambda i: (i, 0, 0)),
        ],
        out_specs=[
            pl.BlockSpec((NUM_GRAPHS, D), lambda i: (0, 0)),
            pl.BlockSpec((NUM_GRAPHS, 128), lambda i: (0, 0)),
        ],
        out_shape=[
            jax.ShapeDtypeStruct((NUM_GRAPHS, D), F32),
            jax.ShapeDtypeStruct((NUM_GRAPHS, 128), F32),
        ],
    )(agg, s, t, batch3d)


def _head_body(p_ref, c_ref, w_ref, b_ref, o_ref):
    cnt = jnp.maximum(c_ref[...][:, 0:1], 1.0)
    o_ref[...] = jnp.dot(p_ref[...] / cnt, w_ref[...], preferred_element_type=F32) + b_ref[...]


def _head(pooled, cnt, w, b):
    return pl.pallas_call(
        _head_body,
        out_shape=jax.ShapeDtypeStruct((NUM_GRAPHS, w.shape[1]), F32),
    )(pooled, cnt, w, b)


def _sc_agg_body(hp, dw, cc, xw_hbm, ad_hbm, ae_hbm, src_hbm, dst_hbm, rp_hbm,
                 out_hbm, rows_v, acc_v, adv, ae_v, idx_v, dst_v, rp_v,
                 sem0, sem1):
    nh = hp // 16
    njf = D // 16
    njt = njf + 2 * nh
    sems = (sem0, sem1)
    wid = lax.axis_index("c") * 16 + lax.axis_index("s")
    base_n = pl.multiple_of(wid * NT, 8)
    pltpu.sync_copy(rp_hbm.at[pl.ds(base_n, NT + 32)], rp_v)

    def chunk_body(c, _):
        nbl = c * CN
        nb = base_n + nbl

        def zero_body(i, _):
            z16 = jnp.zeros((16,), F32)
            for j in range(njt):
                acc_v[i, pl.ds(j * 16, 16)] = z16
            return 0

        lax.fori_loop(0, CN, zero_body, 0)
        pltpu.sync_copy(ad_hbm.at[pl.ds(nb, CN)], adv)
        e0 = rp_v[pl.ds(nbl, 16)][0]
        e1 = rp_v[pl.ds(nbl + CN, 16)][0]
        base_e = e0 - lax.rem(e0, 8)
        nbat = lax.div(e1 - base_e + (EB - 1), EB)

        def issue(b, half):
            ebase = pl.multiple_of(base_e + b * EB, 8)
            ho = half * EB
            pltpu.sync_copy(src_hbm.at[pl.ds(ebase, EB)],
                            idx_v.at[pl.ds(ho, EB)])
            pltpu.sync_copy(dst_hbm.at[pl.ds(ebase, EB)],
                            dst_v.at[pl.ds(ho, EB)])
            pltpu.sync_copy(ae_hbm.at[pl.ds(ebase, EB)],
                            ae_v.at[pl.ds(ho, EB)])
            pltpu.async_copy(xw_hbm.at[idx_v.at[pl.ds(ho, EB)]],
                             rows_v.at[pl.ds(ho, EB)], sems[half])

        def wait(half):
            ho = half * EB
            pltpu.make_async_copy(xw_hbm.at[idx_v.at[pl.ds(ho, EB)]],
                                  rows_v.at[pl.ds(ho, EB)], sems[half]).wait()

        def process(b, half):
            ebase = pl.multiple_of(base_e + b * EB, 8)
            lo = jnp.maximum(e0, ebase) - ebase
            hi = jnp.minimum(e1, ebase + EB) - ebase
            b0 = half * EB

            def loop_a(i, _):
                r = b0 + i
                dl = dst_v[pl.ds(r, 16)][0] - nb
                for k in range(nh):
                    a = rows_v[r, pl.ds(D + k * 16, 16)]
                    ad_k = adv[dl, pl.ds(k * 16, 16)]
                    ae_k = ae_v[r, pl.ds(k * 16, 16)]
                    lg = a + ad_k + ae_k
                    lg = jnp.where(lg < 0, lg * 0.2, lg)
                    w = jnp.exp(lg)
                    rows_v[r, pl.ds(D + k * 16, 16)] = w
                    rows_v[r, pl.ds(D + hp + k * 16, 16)] = ae_k
                return 0

            lax.fori_loop(lo, hi, loop_a, 0)

            nfirst = dst_v[pl.ds(b0 + lo, 16)][0] - nb
            nlast = dst_v[pl.ds(b0 + hi - 1, 16)][0] - nb

            def loop_b(n, _):
                rpv = rp_v[pl.ds(nbl + n, 16)]
                il = jnp.maximum(rpv[0], ebase + lo) - ebase
                ih = jnp.minimum(rpv[1], ebase + hi) - ebase
                for go in range(0, njt, 13):
                    gl = min(13, njt - go)
                    vs = tuple(acc_v[n, pl.ds((go + q) * 16, 16)]
                               for q in range(gl))

                    def sum_body(i, vv):
                        r = b0 + i
                        wv = {}
                        out = []
                        for q, v in enumerate(vv):
                            j = go + q
                            row = rows_v[r, pl.ds(j * 16, 16)]
                            if j < njf:
                                h = (j * 16) // cc
                                k = h // 16
                                if k not in wv:
                                    wv[k] = rows_v[r, pl.ds(D + k * 16, 16)]
                                out.append(v + wv[k][h % 16] * row)
                            else:
                                out.append(v + row)
                        return tuple(out)

                    vs = lax.fori_loop(il, ih, sum_body, vs)
                    for q in range(gl):
                        acc_v[n, pl.ds((go + q) * 16, 16)] = vs[q]
                return 0

            lax.fori_loop(nfirst, nlast + 1, loop_b, 0)

        @pl.when(nbat > 0)
        def _():
            issue(0, 0)

        def pair_body(p, _):
            b0 = 2 * p

            @pl.when(b0 + 1 < nbat)
            def _():
                issue(b0 + 1, 1)

            wait(0)
            process(b0, 0)

            @pl.when(b0 + 1 < nbat)
            def _():
                @pl.when(b0 + 2 < nbat)
                def _():
                    issue(b0 + 2, 0)

                wait(1)
                process(b0 + 1, 1)

            return 0

        lax.fori_loop(0, lax.div(nbat + 1, 2), pair_body, 0)

        for g in range(CN // 16):
            pltpu.sync_copy(xw_hbm.at[pl.ds(nb + g * 16, 16)],
                            rows_v.at[pl.ds(0, 16)])

            def node_body(u, _):
                i = g * 16 + u
                rp2 = rp_v[pl.ds(nbl + i, 16)]
                deg = rp2[1] - rp2[0]
                invd = 1.0 / jnp.maximum(jnp.full((16,), deg.astype(F32)), 1.0)
                ws = []
                rs = []
                for k in range(nh):
                    a = rows_v[u, pl.ds(D + k * 16, 16)]
                    ad_k = adv[i, pl.ds(k * 16, 16)]
                    ael = acc_v[i, pl.ds(D + hp + k * 16, 16)] * invd
                    lg = a + ad_k + ael
                    lg = jnp.where(lg < 0, lg * 0.2, lg)
                    w = jnp.exp(lg)
                    ws.append(w)
                    rs.append(1.0 / (acc_v[i, pl.ds(D + k * 16, 16)] + w))
                for j in range(njf):
                    h = (j * 16) // cc
                    wsc = ws[h // 16][h % 16]
                    rsc = rs[h // 16][h % 16]
                    acc_v[i, pl.ds(j * 16, 16)] = (
                        acc_v[i, pl.ds(j * 16, 16)]
                        + wsc * rows_v[u, pl.ds(j * 16, 16)]) * rsc
                return 0

            lax.fori_loop(0, 16, node_body, 0)

        pltpu.sync_copy(acc_v.at[:, pl.ds(0, D)], out_hbm.at[pl.ds(nb, CN)])
        return 0

    lax.fori_loop(0, NCH, chunk_body, 0)


def _sc_agg(xw_cat, ad, ae, srcs, dsts, row_ptr, hp, cc):
    dw = DW
    mesh = plsc.VectorSubcoreMesh(core_axis_name="c", subcore_axis_name="s",
                                  num_cores=2, num_subcores=16)
    fn = pl.kernel(
        functools.partial(_sc_agg_body, hp, dw, cc),
        out_type=jax.ShapeDtypeStruct((NP, D), F32),
        mesh=mesh,
        scratch_types=[
            pltpu.VMEM((2 * EB, dw), F32),
            pltpu.VMEM((CN, D + 2 * hp), F32),
            pltpu.VMEM((CN, hp), F32),
            pltpu.VMEM((2 * EB, hp), F32),
            pltpu.VMEM((2 * EB,), I32),
            pltpu.VMEM((2 * EB + 16,), I32),
            pltpu.VMEM((NT + 32,), I32),
            pltpu.SemaphoreType.DMA,
            pltpu.SemaphoreType.DMA,
        ],
    )
    return fn(xw_cat, ad, ae, srcs, dsts, row_ptr)


def _fold_weights(p, fin, c, h):
    hp = _hp(h)
    w = p['W']
    ws = jnp.einsum('fhc,hc->fh', w.reshape(fin, h, c), p['att_src'][0])
    wd = jnp.einsum('fhc,hc->fh', w.reshape(fin, h, c), p['att_dst'][0])
    wf = jnp.einsum('khc,hc->kh', p['W_edge'].reshape(3, h, c), p['att_edge'][0])
    pad = hp - h
    ws = jnp.pad(ws, ((0, 0), (0, pad)))
    wd = jnp.pad(wd, ((0, 0), (0, pad)))
    wf = jnp.pad(wf, ((0, 0), (0, pad)))
    wcat = jnp.concatenate(
        [w, ws, jnp.zeros((fin, DW - D - hp), F32)], axis=1)
    return wcat, wd, wf


def kernel(x, edge_index, edge_attr, batch, params):
    src = edge_index[0]
    dst = edge_index[1]

    perm = jnp.argsort(dst)
    srcs = jnp.concatenate([src[perm], jnp.zeros((EB,), I32)])
    dsts = jnp.concatenate([dst[perm], jnp.zeros((EB,), I32)])
    ea_s = jnp.concatenate(
        [edge_attr[perm], jnp.zeros((EPAD - N_EDGES, 3), F32)], axis=0)
    row_ptr = jnp.searchsorted(
        dst[perm], jnp.arange(NP + 64, dtype=I32), side='left').astype(I32)

    xin = jnp.pad(x, ((0, NP - N_NODES), (0, 0)))
    batch3d = jnp.pad(batch, (0, NP - N_NODES)).reshape(NP // 512, 1, 512)

    s_aff = jnp.ones((1, x.shape[1]), F32)
    t_aff = jnp.zeros((1, x.shape[1]), F32)
    agg = None
    for li, ((fin, c, h, lin), p) in enumerate(zip(LAYER_DIMS, params['blocks'])):
        hp = _hp(h)
        wcat, wd, wf = _fold_weights(p, fin, c, h)
        xw_cat, ad = _nodemm(xin, wcat, wd, s_aff, t_aff, apply_act=(li > 0))
        ae = _edgemm(ea_s, wf)
        agg = _sc_agg(xw_cat, ad, ae, srcs, dsts, row_ptr, hp, c)
        s0, t0 = _stats(agg, p['bn0_g'][None, :], p['bn0_b'][None, :])
        if lin is not None:
            z = _linmm(agg, s0, t0, p['lin_w'], p['lin_b'][None, :])
            s_aff, t_aff = _stats(z, p['bn1_g'][None, :], p['bn1_b'][None, :])
            xin = z
        else:
            pooled, cnt = _pool(agg, s0, t0, batch3d)
            return _head(pooled, cnt, params['head_w'], params['head_b'][None, :])


# speedup vs baseline: 23.8511x; 1.0032x over previous
import functools

import jax
import jax.numpy as jnp
from jax import lax
from jax.experimental import pallas as pl
from jax.experimental.pallas import tpu as pltpu
from jax.experimental.pallas import tpu_sc as plsc

N_NODES = 10000
N_EDGES = 160000
NUM_GRAPHS = 64
D = 768
LAYER_DIMS = [(128, 16, 48, 16), (16, 32, 24, 32), (32, 64, 12, 64), (64, 128, 6, 128), (128, 256, 3, None)]

NW = 32
NT = 320
NP = NW * NT
CN = 32
NCH = NT // CN
EB = 40
DW = 896
EPAD = ((N_EDGES + EB) // 1024 + 1) * 1024

F32 = jnp.float32
I32 = jnp.int32


def _hp(h):
    return ((h + 15) // 16) * 16


def _nodemm_body(apply_act, fin, x_ref, wcat_ref, wd_ref, s_ref, t_ref, xw_ref, ad_ref):
    i = pl.program_id(0)
    x = x_ref[...]
    if apply_act:
        x = jnp.maximum(x * s_ref[...] + t_ref[...], 0.0)
    rid = i * x.shape[0] + lax.broadcasted_iota(I32, (x.shape[0], 1), 0)
    x = jnp.where(rid < N_NODES, x, 0.0)
    xw_ref[...] = jnp.dot(x, wcat_ref[...], preferred_element_type=F32)
    ad_ref[...] = jnp.dot(x, wd_ref[...], preferred_element_type=F32)


def _nodemm(x, wcat, wd, s, t, apply_act):
    fin = x.shape[1]
    dw = wcat.shape[1]
    hp = wd.shape[1]
    bn = 256
    return pl.pallas_call(
        functools.partial(_nodemm_body, apply_act, fin),
        grid=(NP // bn,),
        in_specs=[
            pl.BlockSpec((bn, fin), lambda i: (i, 0)),
            pl.BlockSpec((fin, dw), lambda i: (0, 0)),
            pl.BlockSpec((fin, hp), lambda i: (0, 0)),
            pl.BlockSpec((1, fin), lambda i: (0, 0)),
            pl.BlockSpec((1, fin), lambda i: (0, 0)),
        ],
        out_specs=[
            pl.BlockSpec((bn, dw), lambda i: (i, 0)),
            pl.BlockSpec((bn, hp), lambda i: (i, 0)),
        ],
        out_shape=[
            jax.ShapeDtypeStruct((NP, dw), F32),
            jax.ShapeDtypeStruct((NP, hp), F32),
        ],
    )(x, wcat, wd, s, t)


def _edgemm_body(ea_ref, wf_ref, ae_ref):
    ae_ref[...] = jnp.dot(ea_ref[...], wf_ref[...], preferred_element_type=F32)


def _edgemm(ea, wf):
    hp = wf.shape[1]
    bn = 1024
    return pl.pallas_call(
        _edgemm_body,
        grid=(EPAD // bn,),
        in_specs=[
            pl.BlockSpec((bn, 3), lambda i: (i, 0)),
            pl.BlockSpec((3, hp), lambda i: (0, 0)),
        ],
        out_specs=pl.BlockSpec((bn, hp), lambda i: (i, 0)),
        out_shape=jax.ShapeDtypeStruct((EPAD, hp), F32),
    )(ea, wf)


def _stats_body(nblk, arr_ref, g_ref, b_ref, s_ref, t_ref, acc1, acc2):
    i = pl.program_id(0)

    @pl.when(i == 0)
    def _():
        acc1[...] = jnp.zeros_like(acc1)
        acc2[...] = jnp.zeros_like(acc2)

    a = arr_ref[...]
    acc1[...] += jnp.sum(a, axis=0, keepdims=True)
    acc2[...] += jnp.sum(a * a, axis=0, keepdims=True)

    @pl.when(i == nblk - 1)
    def _():
        n = jnp.float32(N_NODES)
        mean = acc1[...] / n
        var = acc2[...] / n - mean * mean
        s = g_ref[...] * lax.rsqrt(var + 1e-5)
        s_ref[...] = s
        t_ref[...] = b_ref[...] - mean * s


def _stats(arr, g, b):
    dx = arr.shape[1]
    bn = 512
    nblk = NP // bn
    return pl.pallas_call(
        functools.partial(_stats_body, nblk),
        grid=(nblk,),
        in_specs=[
            pl.BlockSpec((bn, dx), lambda i: (i, 0)),
            pl.BlockSpec((1, dx), lambda i: (0, 0)),
            pl.BlockSpec((1, dx), lambda i: (0, 0)),
        ],
        out_specs=[
            pl.BlockSpec((1, dx), lambda i: (0, 0)),
            pl.BlockSpec((1, dx), lambda i: (0, 0)),
        ],
        out_shape=[
            jax.ShapeDtypeStruct((1, dx), F32),
            jax.ShapeDtypeStruct((1, dx), F32),
        ],
        scratch_shapes=[
            pltpu.VMEM((1, dx), F32),
            pltpu.VMEM((1, dx), F32),
        ],
    )(arr, g, b)


def _linmm_body(agg_ref, s_ref, t_ref, w_ref, b_ref, z_ref):
    i = pl.program_id(0)
    h = jnp.maximum(agg_ref[...] * s_ref[...] + t_ref[...], 0.0)
    rid = i * h.shape[0] + lax.broadcasted_iota(I32, (h.shape[0], 1), 0)
    h = jnp.where(rid < N_NODES, h, 0.0)
    z = jnp.dot(h, w_ref[...], preferred_element_type=F32) + b_ref[...]
    z_ref[...] = jnp.where(rid < N_NODES, z, 0.0)


def _linmm(agg, s, t, w, b):
    lin = w.shape[1]
    bn = 512
    return pl.pallas_call(
        _linmm_body,
        grid=(NP // bn,),
        in_specs=[
            pl.BlockSpec((bn, D), lambda i: (i, 0)),
            pl.BlockSpec((1, D), lambda i: (0, 0)),
            pl.BlockSpec((1, D), lambda i: (0, 0)),
            pl.BlockSpec((D, lin), lambda i: (0, 0)),
            pl.BlockSpec((1, lin), lambda i: (0, 0)),
        ],
        out_specs=pl.BlockSpec((bn, lin), lambda i: (i, 0)),
        out_shape=jax.ShapeDtypeStruct((NP, lin), F32),
    )(agg, s, t, w, b)


def _pool_body(agg_ref, s_ref, t_ref, batch_ref, p_ref, c_ref):
    i = pl.program_id(0)

    @pl.when(i == 0)
    def _():
        p_ref[...] = jnp.zeros_like(p_ref)
        c_ref[...] = jnp.zeros_like(c_ref)

    x = jnp.maximum(agg_ref[...] * s_ref[...] + t_ref[...], 0.0)
    bn = x.shape[0]
    rid = i * bn + lax.broadcasted_iota(I32, (bn, 1), 0)
    valid = rid < N_NODES
    x = jnp.where(valid, x, 0.0)
    bvec = batch_ref[0, 0, :]
    gids = lax.broadcasted_iota(I32, (NUM_GRAPHS, bn), 0)
    oh = jnp.where((bvec[None, :] == gids) & (valid[:, 0])[None, :], 1.0, 0.0)
    p_ref[...] += jnp.dot(oh, x, preferred_element_type=F32)
    c_ref[...] += jnp.dot(oh, jnp.ones((bn, 128), F32), preferred_element_type=F32)


def _pool(agg, s, t, batch3d):
    bn = 512
    return pl.pallas_call(
        _pool_body,
        grid=(NP // bn,),
        in_specs=[
            pl.BlockSpec((bn, D), lambda i: (i, 0)),
            pl.BlockSpec((1, D), lambda i: (0, 0)),
            pl.BlockSpec((1, D), lambda i: (0, 0)),
            pl.BlockSpec((1, 1, bn), lambda i: (i, 0, 0)),
        ],
        out_specs=[
            pl.BlockSpec((NUM_GRAPHS, D), lambda i: (0, 0)),
            pl.BlockSpec((NUM_GRAPHS, 128), lambda i: (0, 0)),
        ],
        out_shape=[
            jax.ShapeDtypeStruct((NUM_GRAPHS, D), F32),
            jax.ShapeDtypeStruct((NUM_GRAPHS, 128), F32),
        ],
    )(agg, s, t, batch3d)


def _head_body(p_ref, c_ref, w_ref, b_ref, o_ref):
    cnt = jnp.maximum(c_ref[...][:, 0:1], 1.0)
    o_ref[...] = jnp.dot(p_ref[...] / cnt, w_ref[...], preferred_element_type=F32) + b_ref[...]


def _head(pooled, cnt, w, b):
    return pl.pallas_call(
        _head_body,
        out_shape=jax.ShapeDtypeStruct((NUM_GRAPHS, w.shape[1]), F32),
    )(pooled, cnt, w, b)


def _sc_agg_body(hp, dw, cc, xw_hbm, ad_hbm, ae_hbm, src_hbm, dst_hbm, rp_hbm,
                 out_hbm, rows_v, acc_v, adv, ae_v, idx_v, dst_v, rp_v,
                 sem0, sem1):
    nh = hp // 16
    njf = D // 16
    njt = njf + 2 * nh
    sems = (sem0, sem1)
    wid = lax.axis_index("c") * 16 + lax.axis_index("s")
    base_n = pl.multiple_of(wid * NT, 8)
    pltpu.sync_copy(rp_hbm.at[pl.ds(base_n, NT + 32)], rp_v)

    def chunk_body(c, _):
        nbl = c * CN
        nb = base_n + nbl

        def zero_body(i, _):
            z16 = jnp.zeros((16,), F32)
            for j in range(njt):
                acc_v[i, pl.ds(j * 16, 16)] = z16
            return 0

        lax.fori_loop(0, CN, zero_body, 0)
        pltpu.sync_copy(ad_hbm.at[pl.ds(nb, CN)], adv)
        e0 = rp_v[pl.ds(nbl, 16)][0]
        e1 = rp_v[pl.ds(nbl + CN, 16)][0]
        base_e = e0 - lax.rem(e0, 8)
        nbat = lax.div(e1 - base_e + (EB - 1), EB)

        def issue(b, half):
            ebase = pl.multiple_of(base_e + b * EB, 8)
            ho = half * EB
            pltpu.sync_copy(src_hbm.at[pl.ds(ebase, EB)],
                            idx_v.at[pl.ds(ho, EB)])
            pltpu.sync_copy(dst_hbm.at[pl.ds(ebase, EB)],
                            dst_v.at[pl.ds(ho, EB)])
            pltpu.sync_copy(ae_hbm.at[pl.ds(ebase, EB)],
                            ae_v.at[pl.ds(ho, EB)])
            pltpu.async_copy(xw_hbm.at[idx_v.at[pl.ds(ho, EB)]],
                             rows_v.at[pl.ds(ho, EB)], sems[half])

        def wait(half):
            ho = half * EB
            pltpu.make_async_copy(xw_hbm.at[idx_v.at[pl.ds(ho, EB)]],
                                  rows_v.at[pl.ds(ho, EB)], sems[half]).wait()

        def process(b, half):
            ebase = pl.multiple_of(base_e + b * EB, 8)
            lo = jnp.maximum(e0, ebase) - ebase
            hi = jnp.minimum(e1, ebase + EB) - ebase
            b0 = half * EB

            def loop_a(i, _):
                r = b0 + i
                dl = dst_v[pl.ds(r, 16)][0] - nb
                for k in range(nh):
                    a = rows_v[r, pl.ds(D + k * 16, 16)]
                    ad_k = adv[dl, pl.ds(k * 16, 16)]
                    ae_k = ae_v[r, pl.ds(k * 16, 16)]
                    lg = a + ad_k + ae_k
                    lg = jnp.where(lg < 0, lg * 0.2, lg)
                    w = jnp.exp(lg)
                    rows_v[r, pl.ds(D + k * 16, 16)] = w
                    rows_v[r, pl.ds(D + hp + k * 16, 16)] = ae_k
                return 0

            lax.fori_loop(lo, hi, loop_a, 0)

            nfirst = dst_v[pl.ds(b0 + lo, 16)][0] - nb
            nlast = dst_v[pl.ds(b0 + hi - 1, 16)][0] - nb

            def loop_b(n, _):
                rpv = rp_v[pl.ds(nbl + n, 16)]
                il = jnp.maximum(rpv[0], ebase + lo) - ebase
                ih = jnp.minimum(rpv[1], ebase + hi) - ebase
                for go in range(0, njt, 17):
                    gl = min(17, njt - go)
                    vs = tuple(acc_v[n, pl.ds((go + q) * 16, 16)]
                               for q in range(gl))

                    def sum_body(i, vv):
                        r = b0 + i
                        wv = {}
                        out = []
                        for q, v in enumerate(vv):
                            j = go + q
                            row = rows_v[r, pl.ds(j * 16, 16)]
                            if j < njf:
                                h = (j * 16) // cc
                                k = h // 16
                                if k not in wv:
                                    wv[k] = rows_v[r, pl.ds(D + k * 16, 16)]
                                out.append(v + wv[k][h % 16] * row)
                            else:
                                out.append(v + row)
                        return tuple(out)

                    vs = lax.fori_loop(il, ih, sum_body, vs)
                    for q in range(gl):
                        acc_v[n, pl.ds((go + q) * 16, 16)] = vs[q]
                return 0

            lax.fori_loop(nfirst, nlast + 1, loop_b, 0)

        @pl.when(nbat > 0)
        def _():
            issue(0, 0)

        def pair_body(p, _):
            b0 = 2 * p

            @pl.when(b0 + 1 < nbat)
            def _():
                issue(b0 + 1, 1)

            wait(0)
            process(b0, 0)

            @pl.when(b0 + 1 < nbat)
            def _():
                @pl.when(b0 + 2 < nbat)
                def _():
                    issue(b0 + 2, 0)

                wait(1)
                process(b0 + 1, 1)

            return 0

        lax.fori_loop(0, lax.div(nbat + 1, 2), pair_body, 0)

        for g in range(CN // 16):
            pltpu.sync_copy(xw_hbm.at[pl.ds(nb + g * 16, 16)],
                            rows_v.at[pl.ds(0, 16)])

            def node_body(u, _):
                i = g * 16 + u
                rp2 = rp_v[pl.ds(nbl + i, 16)]
                deg = rp2[1] - rp2[0]
                invd = 1.0 / jnp.maximum(jnp.full((16,), deg.astype(F32)), 1.0)
                ws = []
                rs = []
                for k in range(nh):
                    a = rows_v[u, pl.ds(D + k * 16, 16)]
                    ad_k = adv[i, pl.ds(k * 16, 16)]
                    ael = acc_v[i, pl.ds(D + hp + k * 16, 16)] * invd
                    lg = a + ad_k + ael
                    lg = jnp.where(lg < 0, lg * 0.2, lg)
                    w = jnp.exp(lg)
                    ws.append(w)
                    rs.append(1.0 / (acc_v[i, pl.ds(D + k * 16, 16)] + w))
                for j in range(njf):
                    h = (j * 16) // cc
                    wsc = ws[h // 16][h % 16]
                    rsc = rs[h // 16][h % 16]
                    acc_v[i, pl.ds(j * 16, 16)] = (
                        acc_v[i, pl.ds(j * 16, 16)]
                        + wsc * rows_v[u, pl.ds(j * 16, 16)]) * rsc
                return 0

            lax.fori_loop(0, 16, node_body, 0)

        pltpu.sync_copy(acc_v.at[:, pl.ds(0, D)], out_hbm.at[pl.ds(nb, CN)])
        return 0

    lax.fori_loop(0, NCH, chunk_body, 0)


def _sc_agg(xw_cat, ad, ae, srcs, dsts, row_ptr, hp, cc):
    dw = DW
    mesh = plsc.VectorSubcoreMesh(core_axis_name="c", subcore_axis_name="s",
                                  num_cores=2, num_subcores=16)
    fn = pl.kernel(
        functools.partial(_sc_agg_body, hp, dw, cc),
        out_type=jax.ShapeDtypeStruct((NP, D), F32),
        mesh=mesh,
        scratch_types=[
            pltpu.VMEM((2 * EB, dw), F32),
            pltpu.VMEM((CN, D + 2 * hp), F32),
            pltpu.VMEM((CN, hp), F32),
            pltpu.VMEM((2 * EB, hp), F32),
            pltpu.VMEM((2 * EB,), I32),
            pltpu.VMEM((2 * EB + 16,), I32),
            pltpu.VMEM((NT + 32,), I32),
            pltpu.SemaphoreType.DMA,
            pltpu.SemaphoreType.DMA,
        ],
    )
    return fn(xw_cat, ad, ae, srcs, dsts, row_ptr)


def _fold_weights(p, fin, c, h):
    hp = _hp(h)
    w = p['W']
    ws = jnp.einsum('fhc,hc->fh', w.reshape(fin, h, c), p['att_src'][0])
    wd = jnp.einsum('fhc,hc->fh', w.reshape(fin, h, c), p['att_dst'][0])
    wf = jnp.einsum('khc,hc->kh', p['W_edge'].reshape(3, h, c), p['att_edge'][0])
    pad = hp - h
    ws = jnp.pad(ws, ((0, 0), (0, pad)))
    wd = jnp.pad(wd, ((0, 0), (0, pad)))
    wf = jnp.pad(wf, ((0, 0), (0, pad)))
    wcat = jnp.concatenate(
        [w, ws, jnp.zeros((fin, DW - D - hp), F32)], axis=1)
    return wcat, wd, wf


def kernel(x, edge_index, edge_attr, batch, params):
    src = edge_index[0]
    dst = edge_index[1]

    perm = jnp.argsort(dst)
    srcs = jnp.concatenate([src[perm], jnp.zeros((EB,), I32)])
    dsts = jnp.concatenate([dst[perm], jnp.zeros((EB,), I32)])
    ea_s = jnp.concatenate(
        [edge_attr[perm], jnp.zeros((EPAD - N_EDGES, 3), F32)], axis=0)
    row_ptr = jnp.searchsorted(
        dst[perm], jnp.arange(NP + 64, dtype=I32), side='left').astype(I32)

    xin = jnp.pad(x, ((0, NP - N_NODES), (0, 0)))
    batch3d = jnp.pad(batch, (0, NP - N_NODES)).reshape(NP // 512, 1, 512)

    s_aff = jnp.ones((1, x.shape[1]), F32)
    t_aff = jnp.zeros((1, x.shape[1]), F32)
    agg = None
    for li, ((fin, c, h, lin), p) in enumerate(zip(LAYER_DIMS, params['blocks'])):
        hp = _hp(h)
        wcat, wd, wf = _fold_weights(p, fin, c, h)
        xw_cat, ad = _nodemm(xin, wcat, wd, s_aff, t_aff, apply_act=(li > 0))
        ae = _edgemm(ea_s, wf)
        agg = _sc_agg(xw_cat, ad, ae, srcs, dsts, row_ptr, hp, c)
        s0, t0 = _stats(agg, p['bn0_g'][None, :], p['bn0_b'][None, :])
        if lin is not None:
            z = _linmm(agg, s0, t0, p['lin_w'], p['lin_b'][None, :])
            s_aff, t_aff = _stats(z, p['bn1_g'][None, :], p['bn1_b'][None, :])
            xin = z
        else:
            pooled, cnt = _pool(agg, s0, t0, batch3d)
            return _head(pooled, cnt, params['head_w'], params['head_b'][None, :])
